# baseline (speedup 1.0000x reference)
import functools

import jax
import jax.numpy as jnp
import numpy as np
from jax import lax
from jax.experimental import pallas as pl
from jax.experimental.pallas import tpu as pltpu
from jax.experimental.pallas import tpu_sc as plsc

K_NN = 6
BLK = 1024
BLKC = 1024


def _knn_proj_kernel(pos_ref, pos_t_ref, h_ref, wqt_ref, bq_ref, wkt_ref,
                     bk_ref, wvt_ref, bv_ref, q_ref, kv_ref, idx_ref):
    b = pl.program_id(0)
    n = pos_t_ref.shape[2]

    pos_blk = pos_ref[0]
    pos_t = pos_t_ref[0]

    x2_blk = jnp.sum(pos_blk * pos_blk, axis=1, keepdims=True)
    x2_all = jnp.sum(pos_t * pos_t, axis=0, keepdims=True)

    pb = pos_blk.astype(jnp.bfloat16).astype(jnp.float32)
    pt = pos_t.astype(jnp.bfloat16).astype(jnp.float32)
    s = (pb[:, 0:1] * pt[0:1, :]
         + pb[:, 1:2] * pt[1:2, :]
         + pb[:, 2:3] * pt[2:3, :])
    d2 = x2_blk + x2_all - 2.0 * s
    dists = jnp.sqrt(jnp.maximum(d2, 0.0))

    iota_f = jax.lax.broadcasted_iota(
        jnp.int32, dists.shape, 1).astype(jnp.float32)
    inf = jnp.float32(np.inf)
    work = dists
    amins = []
    for _ in range(K_NN):
        m = jnp.min(work, axis=1, keepdims=True)
        cand = jnp.where(work == m, iota_f, jnp.float32(4096.0))
        amin = jnp.min(cand, axis=1, keepdims=True)
        amins.append(amin)
        work = jnp.where(cand == amin, inf, work)
    idx_blk = jnp.concatenate(amins, axis=1).astype(jnp.int32)
    idx_ref[0] = idx_blk + b * n

    h_blk = h_ref[0]
    q = jnp.dot(h_blk, wqt_ref[...], preferred_element_type=jnp.float32)
    q_ref[0] = q + bq_ref[...]
    k = jnp.dot(h_blk, wkt_ref[...], preferred_element_type=jnp.float32)
    k = k + bk_ref[...]
    v = jnp.dot(h_blk, wvt_ref[...], preferred_element_type=jnp.float32)
    v = v + bv_ref[...]
    kv_ref[0] = jnp.concatenate([k, v], axis=1)


def _attn_kernel(q_ref, *refs):
    kv_refs, out_ref = refs[:K_NN], refs[K_NN]
    d = q_ref.shape[1]
    q = q_ref[...]
    logits = []
    for j in range(K_NN):
        kg = kv_refs[j][:, 0:d]
        logits.append(jnp.sum(q * kg, axis=1, keepdims=True)
                      * jnp.float32(1.0 / 8.0))
    mx = logits[0]
    for j in range(1, K_NN):
        mx = jnp.maximum(mx, logits[j])
    es = [jnp.exp(l - mx) for l in logits]
    ssum = es[0]
    for j in range(1, K_NN):
        ssum = ssum + es[j]
    out = jnp.zeros(q.shape, jnp.float32)
    for j in range(K_NN):
        vg = kv_refs[j][:, d:2 * d]
        out = out + (es[j] / ssum) * vg
    out_ref[...] = out


def _make_sc_gather(total, d2x):
    info = plsc.get_sparse_core_info()
    nw = info.num_cores * info.num_subcores
    per_w = total // nw
    ch = 128
    n_ch = per_w // ch
    mesh = plsc.VectorSubcoreMesh(core_axis_name="c", subcore_axis_name="s")

    @functools.partial(
        pl.kernel, mesh=mesh,
        out_type=jax.ShapeDtypeStruct((total, d2x), jnp.float32),
        scratch_types=[
            pltpu.VMEM((per_w,), jnp.int32),
            pltpu.VMEM((ch, d2x), jnp.float32),
            pltpu.VMEM((ch, d2x), jnp.float32),
            pltpu.SemaphoreType.DMA,
            pltpu.SemaphoreType.DMA,
        ],
    )
    def gather(kv_hbm, idx_hbm, out_hbm, idx_all, r0, r1, s0, s1):
        wid = lax.axis_index("s") * info.num_cores + lax.axis_index("c")
        base = wid * per_w
        pltpu.sync_copy(idx_hbm.at[pl.ds(base, per_w)], idx_all)
        bufs, sems = (r0, r1), (s0, s1)
        pending = [None, None]
        pending[0] = pltpu.async_copy(
            kv_hbm.at[idx_all.at[pl.ds(0, ch)]], r0, s0)
        for c in range(n_ch):
            if c + 1 < n_ch:
                pending[(c + 1) % 2] = pltpu.async_copy(
                    kv_hbm.at[idx_all.at[pl.ds((c + 1) * ch, ch)]],
                    bufs[(c + 1) % 2], sems[(c + 1) % 2])
            pending[c % 2].wait()
            pltpu.sync_copy(bufs[c % 2], out_hbm.at[pl.ds(base + c * ch, ch)])

    return gather


@jax.jit
def kernel(h, pos, Wq, bq, Wk, bk, Wv, bv):
    B, N, D = h.shape
    pos_t = jnp.transpose(pos, (0, 2, 1))
    total = B * N * K_NN
    gather = _make_sc_gather(total, 2 * D)
    nblk = B * N // BLKC

    def _plane_spec(j):
        return pl.BlockSpec((BLKC, 2 * D), lambda r, j=j: (j * nblk + r, 0))

    stage1 = functools.partial(
        pl.pallas_call,
        _knn_proj_kernel,
        grid=(B, N // BLK),
        in_specs=[
            pl.BlockSpec((1, BLK, 3), lambda b, i: (b, i, 0)),
            pl.BlockSpec((1, 3, N), lambda b, i: (b, 0, 0)),
            pl.BlockSpec((1, BLK, D), lambda b, i: (b, i, 0)),
            pl.BlockSpec((D, D), lambda b, i: (0, 0)),
            pl.BlockSpec((1, D), lambda b, i: (0, 0)),
            pl.BlockSpec((D, D), lambda b, i: (0, 0)),
            pl.BlockSpec((1, D), lambda b, i: (0, 0)),
            pl.BlockSpec((D, D), lambda b, i: (0, 0)),
            pl.BlockSpec((1, D), lambda b, i: (0, 0)),
        ],
        out_specs=[
            pl.BlockSpec((1, BLK, D), lambda b, i: (b, i, 0)),
            pl.BlockSpec((1, BLK, 2 * D), lambda b, i: (b, i, 0)),
            pl.BlockSpec((1, BLK, K_NN), lambda b, i: (b, i, 0)),
        ],
        out_shape=[
            jax.ShapeDtypeStruct((B, N, D), jnp.float32),
            jax.ShapeDtypeStruct((B, N, 2 * D), jnp.float32),
            jax.ShapeDtypeStruct((B, N, K_NN), jnp.int32),
        ],
        compiler_params=pltpu.CompilerParams(
            dimension_semantics=("arbitrary", "arbitrary")),
    )
    stage3 = functools.partial(
        pl.pallas_call,
        _attn_kernel,
        grid=(nblk,),
        in_specs=[pl.BlockSpec((BLKC, D), lambda r: (r, 0))]
        + [_plane_spec(j) for j in range(K_NN)],
        out_specs=pl.BlockSpec((BLKC, D), lambda r: (r, 0)),
        out_shape=jax.ShapeDtypeStruct((B * N, D), jnp.float32),
        compiler_params=pltpu.CompilerParams(
            dimension_semantics=("arbitrary",)),
    )

    wqt, wkt, wvt = Wq.T, Wk.T, Wv.T
    bq2, bk2, bv2 = bq.reshape(1, D), bk.reshape(1, D), bv.reshape(1, D)

    q, kv, idx = stage1()(pos, pos_t, h, wqt, bq2, wkt, bk2, wvt, bv2)
    idx_nm = jnp.transpose(idx, (2, 0, 1)).reshape(total)
    kgvg = gather(kv.reshape(B * N, 2 * D), idx_nm)
    out = stage3()(q.reshape(B * N, D), *([kgvg] * K_NN))
    return out.reshape(B, N, D)


# speedup vs baseline: 1.0641x; 1.0641x over previous
import functools

import jax
import jax.numpy as jnp
import numpy as np
from jax import lax
from jax.experimental import pallas as pl
from jax.experimental.pallas import tpu as pltpu
from jax.experimental.pallas import tpu_sc as plsc

K_NN = 6
BLK = 1024
BLKC = 2048


def _knn_proj_kernel(pos_ref, pos_t_ref, h_ref, wqt_ref, bq_ref, wkt_ref,
                     bk_ref, wvt_ref, bv_ref, q_ref, kv_ref, idx_ref):
    b = pl.program_id(0)
    n = pos_t_ref.shape[2]

    pos_blk = pos_ref[0]
    pos_t = pos_t_ref[0]

    x2_blk = jnp.sum(pos_blk * pos_blk, axis=1, keepdims=True)
    x2_all = jnp.sum(pos_t * pos_t, axis=0, keepdims=True)

    pb = pos_blk.astype(jnp.bfloat16)
    pt = pos_t.astype(jnp.bfloat16)
    s = jax.lax.dot_general(pb, pt, (((1,), (0,)), ((), ())),
                            preferred_element_type=jnp.float32)
    d2 = x2_blk + x2_all - 2.0 * s
    dists = jnp.sqrt(jnp.maximum(d2, 0.0))

    iota_f = jax.lax.broadcasted_iota(
        jnp.int32, dists.shape, 1).astype(jnp.float32)
    inf = jnp.float32(np.inf)
    work = dists
    amins = []
    for _ in range(K_NN):
        m = jnp.min(work, axis=1, keepdims=True)
        cand = jnp.where(work == m, iota_f, jnp.float32(4096.0))
        amin = jnp.min(cand, axis=1, keepdims=True)
        amins.append(amin)
        work = jnp.where(cand == amin, inf, work)
    idx_blk = jnp.concatenate(amins, axis=1).astype(jnp.int32)
    idx_ref[0] = idx_blk + b * n

    h_blk = h_ref[0]
    q = jnp.dot(h_blk, wqt_ref[...], preferred_element_type=jnp.float32)
    q_ref[0] = q + bq_ref[...]
    k = jnp.dot(h_blk, wkt_ref[...], preferred_element_type=jnp.float32)
    k = k + bk_ref[...]
    v = jnp.dot(h_blk, wvt_ref[...], preferred_element_type=jnp.float32)
    v = v + bv_ref[...]
    kv_ref[0] = jnp.concatenate([k, v], axis=1)


def _attn_kernel(q_ref, *refs):
    kv_refs, out_ref = refs[:K_NN], refs[K_NN]
    d = q_ref.shape[1]
    q = q_ref[...]
    logits = []
    for j in range(K_NN):
        kg = kv_refs[j][:, 0:d]
        logits.append(jnp.sum(q * kg, axis=1, keepdims=True)
                      * jnp.float32(1.0 / 8.0))
    mx = logits[0]
    for j in range(1, K_NN):
        mx = jnp.maximum(mx, logits[j])
    es = [jnp.exp(l - mx) for l in logits]
    ssum = es[0]
    for j in range(1, K_NN):
        ssum = ssum + es[j]
    out = jnp.zeros(q.shape, jnp.float32)
    for j in range(K_NN):
        vg = kv_refs[j][:, d:2 * d]
        out = out + (es[j] / ssum) * vg
    out_ref[...] = out


def _make_sc_gather(total, d2x):
    info = plsc.get_sparse_core_info()
    nw = info.num_cores * info.num_subcores
    per_w = total // nw
    ch = 128
    n_ch = per_w // ch
    mesh = plsc.VectorSubcoreMesh(core_axis_name="c", subcore_axis_name="s")

    @functools.partial(
        pl.kernel, mesh=mesh,
        out_type=jax.ShapeDtypeStruct((total, d2x), jnp.float32),
        scratch_types=[
            pltpu.VMEM((per_w,), jnp.int32),
            pltpu.VMEM((ch, d2x), jnp.float32),
            pltpu.VMEM((ch, d2x), jnp.float32),
            pltpu.SemaphoreType.DMA,
            pltpu.SemaphoreType.DMA,
        ],
    )
    def gather(kv_hbm, idx_hbm, out_hbm, idx_all, r0, r1, s0, s1):
        wid = lax.axis_index("s") * info.num_cores + lax.axis_index("c")
        base = wid * per_w
        pltpu.sync_copy(idx_hbm.at[pl.ds(base, per_w)], idx_all)
        bufs, sems = (r0, r1), (s0, s1)
        pending = [None, None]
        pending[0] = pltpu.async_copy(
            kv_hbm.at[idx_all.at[pl.ds(0, ch)]], r0, s0)
        for c in range(n_ch):
            if c + 1 < n_ch:
                pending[(c + 1) % 2] = pltpu.async_copy(
                    kv_hbm.at[idx_all.at[pl.ds((c + 1) * ch, ch)]],
                    bufs[(c + 1) % 2], sems[(c + 1) % 2])
            pending[c % 2].wait()
            pltpu.sync_copy(bufs[c % 2], out_hbm.at[pl.ds(base + c * ch, ch)])

    return gather


@jax.jit
def kernel(h, pos, Wq, bq, Wk, bk, Wv, bv):
    B, N, D = h.shape
    pos_t = jnp.transpose(pos, (0, 2, 1))
    total = B * N * K_NN
    gather = _make_sc_gather(total, 2 * D)
    nblk = B * N // BLKC

    def _plane_spec(j):
        return pl.BlockSpec((BLKC, 2 * D), lambda r, j=j: (j * nblk + r, 0))

    stage1 = functools.partial(
        pl.pallas_call,
        _knn_proj_kernel,
        grid=(B, N // BLK),
        in_specs=[
            pl.BlockSpec((1, BLK, 3), lambda b, i: (b, i, 0)),
            pl.BlockSpec((1, 3, N), lambda b, i: (b, 0, 0)),
            pl.BlockSpec((1, BLK, D), lambda b, i: (b, i, 0)),
            pl.BlockSpec((D, D), lambda b, i: (0, 0)),
            pl.BlockSpec((1, D), lambda b, i: (0, 0)),
            pl.BlockSpec((D, D), lambda b, i: (0, 0)),
            pl.BlockSpec((1, D), lambda b, i: (0, 0)),
            pl.BlockSpec((D, D), lambda b, i: (0, 0)),
            pl.BlockSpec((1, D), lambda b, i: (0, 0)),
        ],
        out_specs=[
            pl.BlockSpec((1, BLK, D), lambda b, i: (b, i, 0)),
            pl.BlockSpec((1, BLK, 2 * D), lambda b, i: (b, i, 0)),
            pl.BlockSpec((1, BLK, K_NN), lambda b, i: (b, i, 0)),
        ],
        out_shape=[
            jax.ShapeDtypeStruct((B, N, D), jnp.float32),
            jax.ShapeDtypeStruct((B, N, 2 * D), jnp.float32),
            jax.ShapeDtypeStruct((B, N, K_NN), jnp.int32),
        ],
        compiler_params=pltpu.CompilerParams(
            dimension_semantics=("arbitrary", "arbitrary")),
    )
    stage3 = functools.partial(
        pl.pallas_call,
        _attn_kernel,
        grid=(nblk,),
        in_specs=[pl.BlockSpec((BLKC, D), lambda r: (r, 0))]
        + [_plane_spec(j) for j in range(K_NN)],
        out_specs=pl.BlockSpec((BLKC, D), lambda r: (r, 0)),
        out_shape=jax.ShapeDtypeStruct((B * N, D), jnp.float32),
        compiler_params=pltpu.CompilerParams(
            dimension_semantics=("arbitrary",)),
    )

    wqt, wkt, wvt = Wq.T, Wk.T, Wv.T
    bq2, bk2, bv2 = bq.reshape(1, D), bk.reshape(1, D), bv.reshape(1, D)

    q, kv, idx = stage1()(pos, pos_t, h, wqt, bq2, wkt, bk2, wvt, bv2)
    idx_nm = jnp.transpose(idx, (2, 0, 1)).reshape(total)
    kgvg = gather(kv.reshape(B * N, 2 * D), idx_nm)
    out = stage3()(q.reshape(B * N, D), *([kgvg] * K_NN))
    return out.reshape(B, N, D)


# speedup vs baseline: 1.0729x; 1.0083x over previous
import functools

import jax
import jax.numpy as jnp
import numpy as np
from jax import lax
from jax.experimental import pallas as pl
from jax.experimental.pallas import tpu as pltpu
from jax.experimental.pallas import tpu_sc as plsc

K_NN = 6
BLK = 512
BLKC = 2048


def _knn_proj_kernel(pos_ref, pos_t_ref, h_ref, wqt_ref, bq_ref, wkt_ref,
                     bk_ref, wvt_ref, bv_ref, q_ref, kv_ref, idx_ref):
    b = pl.program_id(0)
    n = pos_t_ref.shape[2]

    pos_blk = pos_ref[0]
    pos_t = pos_t_ref[0]

    x2_blk = jnp.sum(pos_blk * pos_blk, axis=1, keepdims=True)
    x2_all = jnp.sum(pos_t * pos_t, axis=0, keepdims=True)

    pb = (-2.0 * pos_blk.astype(jnp.bfloat16).astype(jnp.float32)
          ).astype(jnp.bfloat16)
    pt = pos_t.astype(jnp.bfloat16)
    s2 = jax.lax.dot_general(pb, pt, (((1,), (0,)), ((), ())),
                             preferred_element_type=jnp.float32)
    d2 = x2_blk + x2_all + s2
    dists = jnp.sqrt(jnp.maximum(d2, 0.0))

    iota_f = jax.lax.broadcasted_iota(
        jnp.int32, dists.shape, 1).astype(jnp.float32)
    inf = jnp.float32(np.inf)
    work = dists
    amins = []
    for _ in range(K_NN):
        m = jnp.min(work, axis=1, keepdims=True)
        cand = jnp.where(work == m, iota_f, jnp.float32(4096.0))
        amin = jnp.min(cand, axis=1, keepdims=True)
        amins.append(amin)
        work = jnp.where(cand == amin, inf, work)
    idx_blk = jnp.concatenate(amins, axis=1).astype(jnp.int32)
    idx_ref[0] = idx_blk + b * n

    h_blk = h_ref[0]
    q = jnp.dot(h_blk, wqt_ref[...], preferred_element_type=jnp.float32)
    q_ref[0] = q + bq_ref[...]
    k = jnp.dot(h_blk, wkt_ref[...], preferred_element_type=jnp.float32)
    k = k + bk_ref[...]
    v = jnp.dot(h_blk, wvt_ref[...], preferred_element_type=jnp.float32)
    v = v + bv_ref[...]
    kv_ref[0] = jnp.concatenate([k, v], axis=1)


def _attn_kernel(q_ref, *refs):
    kv_refs, out_ref = refs[:K_NN], refs[K_NN]
    d = q_ref.shape[1]
    q = q_ref[...]
    logits = []
    for j in range(K_NN):
        kg = kv_refs[j][:, 0:d]
        logits.append(jnp.sum(q * kg, axis=1, keepdims=True)
                      * jnp.float32(1.0 / 8.0))
    mx = logits[0]
    for j in range(1, K_NN):
        mx = jnp.maximum(mx, logits[j])
    es = [jnp.exp(l - mx) for l in logits]
    ssum = es[0]
    for j in range(1, K_NN):
        ssum = ssum + es[j]
    out = jnp.zeros(q.shape, jnp.float32)
    for j in range(K_NN):
        vg = kv_refs[j][:, d:2 * d]
        out = out + (es[j] / ssum) * vg
    out_ref[...] = out


def _make_sc_gather(total, d2x):
    info = plsc.get_sparse_core_info()
    nw = info.num_cores * info.num_subcores
    per_w = total // nw
    ch = 128
    n_ch = per_w // ch
    mesh = plsc.VectorSubcoreMesh(core_axis_name="c", subcore_axis_name="s")

    @functools.partial(
        pl.kernel, mesh=mesh,
        out_type=jax.ShapeDtypeStruct((total, d2x), jnp.float32),
        scratch_types=[
            pltpu.VMEM((per_w,), jnp.int32),
            pltpu.VMEM((ch, d2x), jnp.float32),
            pltpu.VMEM((ch, d2x), jnp.float32),
            pltpu.SemaphoreType.DMA,
            pltpu.SemaphoreType.DMA,
        ],
    )
    def gather(kv_hbm, idx_hbm, out_hbm, idx_all, r0, r1, s0, s1):
        wid = lax.axis_index("s") * info.num_cores + lax.axis_index("c")
        base = wid * per_w
        pltpu.sync_copy(idx_hbm.at[pl.ds(base, per_w)], idx_all)
        bufs, sems = (r0, r1), (s0, s1)
        pending = [None, None]
        pending[0] = pltpu.async_copy(
            kv_hbm.at[idx_all.at[pl.ds(0, ch)]], r0, s0)
        for c in range(n_ch):
            if c + 1 < n_ch:
                pending[(c + 1) % 2] = pltpu.async_copy(
                    kv_hbm.at[idx_all.at[pl.ds((c + 1) * ch, ch)]],
                    bufs[(c + 1) % 2], sems[(c + 1) % 2])
            pending[c % 2].wait()
            pltpu.sync_copy(bufs[c % 2], out_hbm.at[pl.ds(base + c * ch, ch)])

    return gather


@jax.jit
def kernel(h, pos, Wq, bq, Wk, bk, Wv, bv):
    B, N, D = h.shape
    pos_t = jnp.transpose(pos, (0, 2, 1))
    total = B * N * K_NN
    gather = _make_sc_gather(total, 2 * D)
    nblk = B * N // BLKC

    def _plane_spec(j):
        return pl.BlockSpec((BLKC, 2 * D), lambda r, j=j: (j * nblk + r, 0))

    stage1 = functools.partial(
        pl.pallas_call,
        _knn_proj_kernel,
        grid=(B, N // BLK),
        in_specs=[
            pl.BlockSpec((1, BLK, 3), lambda b, i: (b, i, 0)),
            pl.BlockSpec((1, 3, N), lambda b, i: (b, 0, 0)),
            pl.BlockSpec((1, BLK, D), lambda b, i: (b, i, 0)),
            pl.BlockSpec((D, D), lambda b, i: (0, 0)),
            pl.BlockSpec((1, D), lambda b, i: (0, 0)),
            pl.BlockSpec((D, D), lambda b, i: (0, 0)),
            pl.BlockSpec((1, D), lambda b, i: (0, 0)),
            pl.BlockSpec((D, D), lambda b, i: (0, 0)),
            pl.BlockSpec((1, D), lambda b, i: (0, 0)),
        ],
        out_specs=[
            pl.BlockSpec((1, BLK, D), lambda b, i: (b, i, 0)),
            pl.BlockSpec((1, BLK, 2 * D), lambda b, i: (b, i, 0)),
            pl.BlockSpec((1, BLK, K_NN), lambda b, i: (b, i, 0)),
        ],
        out_shape=[
            jax.ShapeDtypeStruct((B, N, D), jnp.float32),
            jax.ShapeDtypeStruct((B, N, 2 * D), jnp.float32),
            jax.ShapeDtypeStruct((B, N, K_NN), jnp.int32),
        ],
        compiler_params=pltpu.CompilerParams(
            dimension_semantics=("arbitrary", "arbitrary")),
    )
    stage3 = functools.partial(
        pl.pallas_call,
        _attn_kernel,
        grid=(nblk,),
        in_specs=[pl.BlockSpec((BLKC, D), lambda r: (r, 0))]
        + [_plane_spec(j) for j in range(K_NN)],
        out_specs=pl.BlockSpec((BLKC, D), lambda r: (r, 0)),
        out_shape=jax.ShapeDtypeStruct((B * N, D), jnp.float32),
        compiler_params=pltpu.CompilerParams(
            dimension_semantics=("arbitrary",)),
    )

    wqt, wkt, wvt = Wq.T, Wk.T, Wv.T
    bq2, bk2, bv2 = bq.reshape(1, D), bk.reshape(1, D), bv.reshape(1, D)

    q, kv, idx = stage1()(pos, pos_t, h, wqt, bq2, wkt, bk2, wvt, bv2)
    idx_nm = jnp.transpose(idx, (2, 0, 1)).reshape(total)
    kgvg = gather(kv.reshape(B * N, 2 * D), idx_nm)
    out = stage3()(q.reshape(B * N, D), *([kgvg] * K_NN))
    return out.reshape(B, N, D)
